# Initial kernel scaffold; baseline (speedup 1.0000x reference)
import jax
import jax.numpy as jnp
from jax.experimental import pallas as pl


def kernel(H, edge_index, edge_weight, W):
    raise NotImplementedError("write your pallas kernel here")


# speedup vs baseline: 20.3743x; 20.3743x over previous
import functools

import jax
import jax.numpy as jnp
from jax import lax
from jax.experimental import pallas as pl
from jax.experimental.pallas import tpu as pltpu
from jax.experimental.pallas import tpu_sc as plsc

NC = 2
NS = 16
NLANE = 16
NTILES = NC * NS

CHUNK = 128
N_PAD = 10240
BLK = 256


def _round_up(x, m):
    return (x + m - 1) // m * m


def _deg_body(dst_hbm, w_hbm, out_hbm, idx_v, w_v, zbuf, deg_sh, *, rpt):
    c = lax.axis_index("c")
    s = lax.axis_index("s")
    wid = c * NS + s
    per_tile = N_PAD // NS

    def zb(i, _):
        zbuf[pl.ds(i * NLANE, NLANE)] = jnp.zeros((NLANE,), jnp.float32)
        return 0
    lax.fori_loop(0, per_tile // NLANE, zb, 0)
    pltpu.sync_copy(zbuf, deg_sh.at[pl.ds(s * per_tile, per_tile)])
    plsc.subcore_barrier()

    pltpu.sync_copy(dst_hbm.at[pl.ds(wid * rpt, rpt)], idx_v)
    pltpu.sync_copy(w_hbm.at[pl.ds(wid * rpt, rpt)], w_v)

    def row(j, _):
        pltpu.sync_copy(w_v.at[j], deg_sh.at[idx_v.at[j]], add=True)
        return 0
    lax.fori_loop(0, rpt, row, 0)
    plsc.subcore_barrier()

    @pl.when(s == 0)
    def _():
        pltpu.sync_copy(deg_sh, out_hbm.at[c])


def _deg_kernel(dst2d, w2d, rpt):
    mesh = plsc.VectorSubcoreMesh(
        core_axis_name="c", subcore_axis_name="s", num_cores=NC, num_subcores=NS)
    return pl.kernel(
        functools.partial(_deg_body, rpt=rpt),
        out_type=jax.ShapeDtypeStruct((NC, N_PAD), jnp.float32),
        mesh=mesh,
        scratch_types=[
            pltpu.VMEM((rpt, CHUNK), jnp.int32),
            pltpu.VMEM((rpt, CHUNK), jnp.float32),
            pltpu.VMEM((N_PAD // NS,), jnp.float32),
            pltpu.VMEM_SHARED((N_PAD,), jnp.float32),
        ],
    )(dst2d, w2d)


def _gmat_body(h_ref, w_ref, degp_ref, g_ref):
    deg = degp_ref[0, :] + degp_ref[1, :]
    r = lax.rsqrt(jnp.maximum(deg, 1e-6))
    g_ref[...] = lax.dot_general(
        h_ref[...] * r[:, None], w_ref[...],
        (((1,), (1,)), ((), ())), preferred_element_type=jnp.float32)


def _gmat(h_pad, W, deg_parts):
    grid = N_PAD // BLK
    return pl.pallas_call(
        _gmat_body,
        grid=(grid,),
        in_specs=[
            pl.BlockSpec((BLK, 128), lambda i: (i, 0)),
            pl.BlockSpec((128, 128), lambda i: (0, 0)),
            pl.BlockSpec((NC, BLK), lambda i: (0, i)),
        ],
        out_specs=pl.BlockSpec((BLK, 128), lambda i: (i, 0)),
        out_shape=jax.ShapeDtypeStruct((N_PAD, 128), jnp.float32),
    )(h_pad, W, deg_parts)


def _scat_body(g_hbm, src_hbm, dst_hbm, w_hbm, out_hbm,
               src_v, dst_v, w_v, rows_v, zbuf, c_sh, *, rpt):
    c = lax.axis_index("c")
    s = lax.axis_index("s")
    wid = c * NS + s
    per_tile = N_PAD // NS
    zrows = 16
    half = rpt // 2

    def zb(i, _):
        zbuf[i >> 3, pl.ds((i & 7) * NLANE, NLANE)] = jnp.zeros((NLANE,), jnp.float32)
        return 0
    lax.fori_loop(0, zrows * 8, zb, 0)

    def zc(k, _):
        pltpu.sync_copy(zbuf, c_sh.at[pl.ds(s * per_tile + k * zrows, zrows)])
        return 0
    lax.fori_loop(0, per_tile // zrows, zc, 0)
    plsc.subcore_barrier()

    def stage(st, _):
        base = wid * rpt + st * half
        pltpu.sync_copy(src_hbm.at[pl.ds(base, half)], src_v)
        pltpu.sync_copy(dst_hbm.at[pl.ds(base, half)], dst_v)
        pltpu.sync_copy(w_hbm.at[pl.ds(base, half)], w_v)

        def row(j, _):
            pltpu.sync_copy(g_hbm.at[src_v.at[j]], rows_v)

            def egroup(g, _):
                wv = w_v[j, pl.ds(g * NLANE, NLANE)]
                for e in range(NLANE):
                    sc = wv[e]
                    row_i = g * NLANE + e
                    for cc in range(8):
                        sl = pl.ds(cc * NLANE, NLANE)
                        rows_v[row_i, sl] = rows_v[row_i, sl] * sc
                return 0
            lax.fori_loop(0, CHUNK // NLANE, egroup, 0)
            pltpu.sync_copy(rows_v, c_sh.at[dst_v.at[j]], add=True)
            return 0
        lax.fori_loop(0, half, row, 0)
        return 0
    lax.fori_loop(0, 2, stage, 0)
    plsc.subcore_barrier()

    pltpu.sync_copy(c_sh.at[pl.ds(s * per_tile, per_tile)],
                    out_hbm.at[c, pl.ds(s * per_tile, per_tile)])


def _scat_kernel(g, src2d, dst2d, w2d, rpt):
    mesh = plsc.VectorSubcoreMesh(
        core_axis_name="c", subcore_axis_name="s", num_cores=NC, num_subcores=NS)
    return pl.kernel(
        functools.partial(_scat_body, rpt=rpt),
        out_type=jax.ShapeDtypeStruct((NC, N_PAD, 128), jnp.float32),
        mesh=mesh,
        scratch_types=[
            pltpu.VMEM((rpt // 2, CHUNK), jnp.int32),
            pltpu.VMEM((rpt // 2, CHUNK), jnp.int32),
            pltpu.VMEM((rpt // 2, CHUNK), jnp.float32),
            pltpu.VMEM((CHUNK, 128), jnp.float32),
            pltpu.VMEM((16, 128), jnp.float32),
            pltpu.VMEM_SHARED((N_PAD, 128), jnp.float32),
        ],
    )(g, src2d, dst2d, w2d)


def _final_body(cparts_ref, degp_ref, out_ref):
    deg = degp_ref[0, :] + degp_ref[1, :]
    r = lax.rsqrt(jnp.maximum(deg, 1e-6))
    out_ref[...] = (cparts_ref[0] + cparts_ref[1]) * r[:, None]


def _final(cparts, deg_parts):
    grid = N_PAD // BLK
    return pl.pallas_call(
        _final_body,
        grid=(grid,),
        in_specs=[
            pl.BlockSpec((NC, BLK, 128), lambda i: (0, i, 0)),
            pl.BlockSpec((NC, BLK), lambda i: (0, i)),
        ],
        out_specs=pl.BlockSpec((BLK, 128), lambda i: (i, 0)),
        out_shape=jax.ShapeDtypeStruct((N_PAD, 128), jnp.float32),
    )(cparts, deg_parts)


def kernel(H, edge_index, edge_weight, W):
    N, D = H.shape
    E = edge_weight.shape[0]
    src = edge_index[0].astype(jnp.int32)
    dst = edge_index[1].astype(jnp.int32)
    w = edge_weight.astype(jnp.float32)

    ep = _round_up(E, NTILES * CHUNK * 8)
    rpt = ep // (NTILES * CHUNK)
    pad = ep - E
    if pad:
        pad_idx = jnp.arange(pad, dtype=jnp.int32) % N
        src = jnp.concatenate([src, pad_idx])
        dst = jnp.concatenate([dst, pad_idx])
        w = jnp.concatenate([w, jnp.zeros((pad,), jnp.float32)])
    src2d = src.reshape(ep // CHUNK, CHUNK)
    dst2d = dst.reshape(ep // CHUNK, CHUNK)
    w2d = w.reshape(ep // CHUNK, CHUNK)

    h_pad = jnp.concatenate(
        [H, jnp.zeros((N_PAD - N, D), jnp.float32)], axis=0)

    deg_parts = _deg_kernel(dst2d, w2d, rpt)
    g = _gmat(h_pad, W, deg_parts)
    cparts = _scat_kernel(g, src2d, dst2d, w2d, rpt)
    out = _final(cparts, deg_parts)
    return out[:N]
